# baseline (speedup 1.0000x reference)
import functools

import jax
import jax.numpy as jnp
from jax import lax
from jax.experimental import pallas as pl
from jax.experimental.pallas import tpu as pltpu
from jax.experimental.pallas import tpu_sc as plsc

B = 4
N = 2048
K = 1024
D = 256
NSEC = 64


_C = 32
_NCH = N // _C


def _score_rank_body(h_ref, sf_ref, s_ref, rank_ref, s_scr):
    w = lax.dot_general(h_ref[0], sf_ref[0], (((1,), (1,)), ((), ())))
    s = 1.0 / (1.0 + jnp.exp(-jnp.max(w, axis=1)))
    s_ref[0, 0] = s
    s_scr[...] = s.reshape(_NCH, _C)
    s_row = s.reshape(1, N)
    jrow = lax.broadcasted_iota(jnp.int32, (1, N), 1)

    def cbody(c, _):
        sc = s_scr[c].reshape(_C, 1)
        icol = lax.broadcasted_iota(jnp.int32, (_C, 1), 0) + c * _C
        gt = (s_row > sc).astype(jnp.float32)
        eqlt = ((s_row == sc) & (jrow < icol)).astype(jnp.float32)
        rank_ref[0, c] = jnp.sum(gt + eqlt, axis=1).astype(jnp.int32)
        return 0

    lax.fori_loop(0, _NCH, cbody, 0)


def _score_rank_tc(h, section_feature):
    s, rank = pl.pallas_call(
        _score_rank_body,
        grid=(B,),
        in_specs=[
            pl.BlockSpec((1, N, D), lambda b: (b, 0, 0)),
            pl.BlockSpec((1, NSEC, D), lambda b: (b, 0, 0)),
        ],
        out_specs=[
            pl.BlockSpec((1, 1, N), lambda b: (b, 0, 0)),
            pl.BlockSpec((1, _NCH, _C), lambda b: (b, 0, 0)),
        ],
        out_shape=[
            jax.ShapeDtypeStruct((B, 1, N), jnp.float32),
            jax.ShapeDtypeStruct((B, _NCH, _C), jnp.int32),
        ],
        scratch_shapes=[pltpu.VMEM((_NCH, _C), jnp.float32)],
    )(h, section_feature)
    return s.reshape(B * N), rank.reshape(B * N)


_NC = 2
_NS = 16
_NW = _NC * _NS
_RPW = (B * K) // _NW
_CHUNK = 16
_WPB = K // _RPW


def _gather_body(g1_hbm, g2_hbm, hf_hbm, s_hbm, rank_hbm,
                 o1_hbm, o2_hbm, newh_hbm,
                 rank_v, s_v, idx_v, vals_v, hidx_v, hbuf, rbuf, obuf, sem):
    wid = lax.axis_index("s") * _NC + lax.axis_index("c")
    base = wid * _RPW
    b = wid // _WPB
    nodebase = pl.multiple_of(b * N, N)
    pbase = (wid % _WPB) * _RPW

    pltpu.sync_copy(rank_hbm.at[pl.ds(nodebase, N)], rank_v)
    pltpu.sync_copy(s_hbm.at[pl.ds(nodebase, N)], s_v)

    def inv_body(t, _):
        r16 = rank_v[pl.ds(t * 16, 16)]
        i16 = lax.broadcasted_iota(jnp.int32, (16,), 0) + t * 16
        m = r16 < K
        plsc.store_scatter(idx_v, [r16], i16, mask=m)
        plsc.store_scatter(vals_v, [r16], s_v[pl.ds(t * 16, 16)], mask=m)
        return 0

    lax.fori_loop(0, N // 16, inv_body, 0)

    def hid_body(t, _):
        hidx_v[pl.ds(t * 16, 16)] = (
            idx_v[pl.ds(pbase + t * 16, 16)] + nodebase)
        return 0

    lax.fori_loop(0, _RPW // 16, hid_body, 0)

    for ci in range(_RPW // _CHUNK):
        r0 = ci * _CHUNK
        pltpu.async_copy(hf_hbm.at[hidx_v.at[pl.ds(r0, _CHUNK)]],
                         hbuf, sem).wait()

        def hscale_body(r, _):
            val = plsc.load_gather(
                vals_v, [jnp.full((16,), pbase + r0 + r, jnp.int32)])

            def dbody(dd, _):
                hbuf[r, pl.ds(dd * 16, 16)] = hbuf[r, pl.ds(dd * 16, 16)] * val
                return 0

            lax.fori_loop(0, D // 16, dbody, 0)
            return 0

        lax.fori_loop(0, _CHUNK, hscale_body, 0)
        pltpu.sync_copy(hbuf, newh_hbm.at[pl.ds(base + r0, _CHUNK)])

    for g_hbm, o_hbm in ((g1_hbm, o1_hbm), (g2_hbm, o2_hbm)):
        for ci in range(_RPW // _CHUNK):
            r0 = ci * _CHUNK
            pltpu.async_copy(g_hbm.at[hidx_v.at[pl.ds(r0, _CHUNK)]],
                             rbuf, sem).wait()

            def rbody(r, _):
                def qbody(q, _):
                    cols = idx_v[pl.ds(q * 16, 16)]
                    rid = jnp.full((16,), r, jnp.int32)
                    v = plsc.load_gather(rbuf, [rid, cols])
                    obuf[r, pl.ds(q * 16, 16)] = v
                    return 0

                lax.fori_loop(0, K // 16, qbody, 0)
                return 0

            lax.fori_loop(0, _CHUNK, rbody, 0)
            pltpu.sync_copy(obuf, o_hbm.at[pl.ds(base + r0, _CHUNK)])


@functools.cache
def _gather_sc_kernel():
    return functools.partial(
        pl.kernel,
        mesh=plsc.VectorSubcoreMesh(core_axis_name="c", subcore_axis_name="s"),
        out_type=[
            jax.ShapeDtypeStruct((B * K, K), jnp.float32),
            jax.ShapeDtypeStruct((B * K, K), jnp.float32),
            jax.ShapeDtypeStruct((B * K, D), jnp.float32),
        ],
        scratch_types=[
            pltpu.VMEM((N,), jnp.int32),
            pltpu.VMEM((N,), jnp.float32),
            pltpu.VMEM((K,), jnp.int32),
            pltpu.VMEM((K,), jnp.float32),
            pltpu.VMEM((_RPW,), jnp.int32),
            pltpu.VMEM((_CHUNK, D), jnp.float32),
            pltpu.VMEM((_CHUNK, N), jnp.float32),
            pltpu.VMEM((_CHUNK, K), jnp.float32),
            pltpu.SemaphoreType.DMA,
        ],
        compiler_params=pltpu.CompilerParams(
            use_tc_tiling_on_sc=False, needs_layout_passes=False),
    )(_gather_body)


def kernel(g1, g2, h, section_feature):
    s, rank = _score_rank_tc(h, section_feature)
    g1f = g1.reshape(B * N, N)
    g2f = g2.reshape(B * N, N)
    hf = h.reshape(B * N, D)
    o1, o2, newh = _gather_sc_kernel()(g1f, g2f, hf, s, rank)
    return (o1.reshape(B, K, K), o2.reshape(B, K, K), newh.reshape(B, K, D))


# speedup vs baseline: 12.4535x; 12.4535x over previous
import functools

import jax
import jax.numpy as jnp
from jax import lax
from jax.experimental import pallas as pl
from jax.experimental.pallas import tpu as pltpu
from jax.experimental.pallas import tpu_sc as plsc

B = 4
N = 2048
K = 1024
D = 256
NSEC = 64


_RC = 128
_RNCH = N // _RC
_JC = 256


def _score_body(h_ref, sf_ref, s_ref):
    w = lax.dot_general(h_ref[0], sf_ref[0], (((1,), (1,)), ((), ())))
    s_ref[0, 0] = 1.0 / (1.0 + jnp.exp(-jnp.max(w, axis=1)))


def _rank_body(s_ref, schunk_ref, rank_ref):
    c = pl.program_id(1)
    sc = schunk_ref[0, 0].reshape(_RC, 1)
    icol = lax.broadcasted_iota(jnp.int32, (_RC, 1), 0) + c * _RC
    s_all = s_ref[0, 0]
    acc = jnp.zeros((_RC, _JC), jnp.float32)
    for jc in range(N // _JC):
        sj = s_all[jc * _JC:(jc + 1) * _JC].reshape(1, _JC)
        jrow = lax.broadcasted_iota(jnp.int32, (1, _JC), 1) + jc * _JC
        gt = (sj > sc).astype(jnp.float32)
        eqlt = ((sj == sc) & (jrow < icol)).astype(jnp.float32)
        acc = acc + gt + eqlt
    rank_ref[0, 0, 0] = jnp.sum(acc, axis=1).astype(jnp.int32)


def _score_rank_tc(h, section_feature):
    s = pl.pallas_call(
        _score_body,
        grid=(B,),
        in_specs=[
            pl.BlockSpec((1, N, D), lambda b: (b, 0, 0)),
            pl.BlockSpec((1, NSEC, D), lambda b: (b, 0, 0)),
        ],
        out_specs=pl.BlockSpec((1, 1, N), lambda b: (b, 0, 0)),
        out_shape=jax.ShapeDtypeStruct((B, 1, N), jnp.float32),
    )(h, section_feature)
    rank = pl.pallas_call(
        _rank_body,
        grid=(B, _RNCH),
        in_specs=[
            pl.BlockSpec((1, 1, N), lambda b, c: (b, 0, 0)),
            pl.BlockSpec((1, 1, _RC), lambda b, c: (b, 0, c)),
        ],
        out_specs=pl.BlockSpec((1, 1, 1, _RC), lambda b, c: (b, c, 0, 0)),
        out_shape=jax.ShapeDtypeStruct((B, _RNCH, 1, _RC), jnp.int32),
    )(s, s)
    return s.reshape(B * N), rank.reshape(B * N)


_NC = 2
_NS = 16
_NW = _NC * _NS
_RPW = (B * K) // _NW
_CHUNK = 16
_WPB = K // _RPW


def _gather_body(g1_hbm, g2_hbm, hf_hbm, s_hbm, rank_hbm,
                 o1_hbm, o2_hbm, newh_hbm,
                 rank_v, s_v, idx_v, vals_v, hidx_v, hbuf, rbuf, obuf, sem):
    wid = lax.axis_index("s") * _NC + lax.axis_index("c")
    base = wid * _RPW
    b = wid // _WPB
    nodebase = pl.multiple_of(b * N, N)
    pbase = (wid % _WPB) * _RPW

    pltpu.sync_copy(rank_hbm.at[pl.ds(nodebase, N)], rank_v)
    pltpu.sync_copy(s_hbm.at[pl.ds(nodebase, N)], s_v)

    def inv_body(t, _):
        r16 = rank_v[pl.ds(t * 16, 16)]
        i16 = lax.broadcasted_iota(jnp.int32, (16,), 0) + t * 16
        m = r16 < K
        plsc.store_scatter(idx_v, [r16], i16, mask=m)
        plsc.store_scatter(vals_v, [r16], s_v[pl.ds(t * 16, 16)], mask=m)
        return 0

    lax.fori_loop(0, N // 16, inv_body, 0)

    def hid_body(t, _):
        hidx_v[pl.ds(t * 16, 16)] = (
            idx_v[pl.ds(pbase + t * 16, 16)] + nodebase)
        return 0

    lax.fori_loop(0, _RPW // 16, hid_body, 0)

    for ci in range(_RPW // _CHUNK):
        r0 = ci * _CHUNK
        pltpu.async_copy(hf_hbm.at[hidx_v.at[pl.ds(r0, _CHUNK)]],
                         hbuf, sem).wait()

        def hscale_body(r, _):
            val = plsc.load_gather(
                vals_v, [jnp.full((16,), pbase + r0 + r, jnp.int32)])

            def dbody(dd, _):
                hbuf[r, pl.ds(dd * 16, 16)] = hbuf[r, pl.ds(dd * 16, 16)] * val
                return 0

            lax.fori_loop(0, D // 16, dbody, 0)
            return 0

        lax.fori_loop(0, _CHUNK, hscale_body, 0)
        pltpu.sync_copy(hbuf, newh_hbm.at[pl.ds(base + r0, _CHUNK)])

    for g_hbm, o_hbm in ((g1_hbm, o1_hbm), (g2_hbm, o2_hbm)):
        for ci in range(_RPW // _CHUNK):
            r0 = ci * _CHUNK
            pltpu.async_copy(g_hbm.at[hidx_v.at[pl.ds(r0, _CHUNK)]],
                             rbuf, sem).wait()

            def rbody(r, _):
                def qbody(q, _):
                    cols = idx_v[pl.ds(q * 16, 16)]
                    rid = jnp.full((16,), r, jnp.int32)
                    v = plsc.load_gather(rbuf, [rid, cols])
                    obuf[r, pl.ds(q * 16, 16)] = v
                    return 0

                lax.fori_loop(0, K // 16, qbody, 0)
                return 0

            lax.fori_loop(0, _CHUNK, rbody, 0)
            pltpu.sync_copy(obuf, o_hbm.at[pl.ds(base + r0, _CHUNK)])


@functools.cache
def _gather_sc_kernel():
    return functools.partial(
        pl.kernel,
        mesh=plsc.VectorSubcoreMesh(core_axis_name="c", subcore_axis_name="s"),
        out_type=[
            jax.ShapeDtypeStruct((B * K, K), jnp.float32),
            jax.ShapeDtypeStruct((B * K, K), jnp.float32),
            jax.ShapeDtypeStruct((B * K, D), jnp.float32),
        ],
        scratch_types=[
            pltpu.VMEM((N,), jnp.int32),
            pltpu.VMEM((N,), jnp.float32),
            pltpu.VMEM((K,), jnp.int32),
            pltpu.VMEM((K,), jnp.float32),
            pltpu.VMEM((_RPW,), jnp.int32),
            pltpu.VMEM((_CHUNK, D), jnp.float32),
            pltpu.VMEM((_CHUNK, N), jnp.float32),
            pltpu.VMEM((_CHUNK, K), jnp.float32),
            pltpu.SemaphoreType.DMA,
        ],
        compiler_params=pltpu.CompilerParams(
            use_tc_tiling_on_sc=False, needs_layout_passes=False),
    )(_gather_body)


def kernel(g1, g2, h, section_feature):
    s, rank = _score_rank_tc(h, section_feature)
    g1f = g1.reshape(B * N, N)
    g2f = g2.reshape(B * N, N)
    hf = h.reshape(B * N, D)
    o1, o2, newh = _gather_sc_kernel()(g1f, g2f, hf, s, rank)
    return (o1.reshape(B, K, K), o2.reshape(B, K, K), newh.reshape(B, K, D))


# speedup vs baseline: 19.8192x; 1.5914x over previous
import functools

import jax
import jax.numpy as jnp
from jax import lax
from jax.experimental import pallas as pl
from jax.experimental.pallas import tpu as pltpu
from jax.experimental.pallas import tpu_sc as plsc

B = 4
N = 2048
K = 1024
D = 256
NSEC = 64


_RC = 128
_RNCH = N // _RC
_JC = 256


def _score_body(h_ref, sf_ref, s_ref):
    w = lax.dot_general(h_ref[0], sf_ref[0], (((1,), (1,)), ((), ())))
    s_ref[0, 0] = 1.0 / (1.0 + jnp.exp(-jnp.max(w, axis=1)))


def _rank_body(s_ref, schunk_ref, rank_ref):
    c = pl.program_id(1)
    sc = schunk_ref[0, 0].reshape(_RC, 1)
    icol = lax.broadcasted_iota(jnp.int32, (_RC, 1), 0) + c * _RC
    s_all = s_ref[0, 0]
    acc = jnp.zeros((_RC, _JC), jnp.float32)
    for jc in range(N // _JC):
        sj = s_all[jc * _JC:(jc + 1) * _JC].reshape(1, _JC)
        jrow = lax.broadcasted_iota(jnp.int32, (1, _JC), 1) + jc * _JC
        gt = (sj > sc).astype(jnp.float32)
        eqlt = ((sj == sc) & (jrow < icol)).astype(jnp.float32)
        acc = acc + gt + eqlt
    rank_ref[0, 0, 0] = jnp.sum(acc, axis=1).astype(jnp.int32)


def _score_rank_tc(h, section_feature):
    s = pl.pallas_call(
        _score_body,
        grid=(B,),
        in_specs=[
            pl.BlockSpec((1, N, D), lambda b: (b, 0, 0)),
            pl.BlockSpec((1, NSEC, D), lambda b: (b, 0, 0)),
        ],
        out_specs=pl.BlockSpec((1, 1, N), lambda b: (b, 0, 0)),
        out_shape=jax.ShapeDtypeStruct((B, 1, N), jnp.float32),
    )(h, section_feature)
    rank = pl.pallas_call(
        _rank_body,
        grid=(B, _RNCH),
        in_specs=[
            pl.BlockSpec((1, 1, N), lambda b, c: (b, 0, 0)),
            pl.BlockSpec((1, 1, _RC), lambda b, c: (b, 0, c)),
        ],
        out_specs=pl.BlockSpec((1, 1, 1, _RC), lambda b, c: (b, c, 0, 0)),
        out_shape=jax.ShapeDtypeStruct((B, _RNCH, 1, _RC), jnp.int32),
    )(s, s)
    return s.reshape(B * N), rank.reshape(B * N)


_NC = 2
_NS = 16
_NW = _NC * _NS
_RPW = (B * K) // _NW
_CHUNK = 16
_WPB = K // _RPW


def _gather_body(g1_hbm, g2_hbm, hf_hbm, s_hbm, rank_hbm,
                 o1_hbm, o2_hbm, newh_hbm,
                 rank_v, s_v, idx_v, vals_v, hidx_v, hbuf,
                 rbuf0, rbuf1, obuf0, obuf1, sem0, sem1, semo0, semo1):
    wid = lax.axis_index("s") * _NC + lax.axis_index("c")
    base = wid * _RPW
    b = wid // _WPB
    nodebase = pl.multiple_of(b * N, N)
    pbase = (wid % _WPB) * _RPW

    pltpu.sync_copy(rank_hbm.at[pl.ds(nodebase, N)], rank_v)
    pltpu.sync_copy(s_hbm.at[pl.ds(nodebase, N)], s_v)

    def inv_body(t, _):
        r16 = rank_v[pl.ds(t * 16, 16)]
        i16 = lax.broadcasted_iota(jnp.int32, (16,), 0) + t * 16
        m = r16 < K
        plsc.store_scatter(idx_v, [r16], i16, mask=m)
        plsc.store_scatter(vals_v, [r16], s_v[pl.ds(t * 16, 16)], mask=m)
        return 0

    lax.fori_loop(0, N // 16, inv_body, 0)

    def hid_body(t, _):
        hidx_v[pl.ds(t * 16, 16)] = (
            idx_v[pl.ds(pbase + t * 16, 16)] + nodebase)
        return 0

    lax.fori_loop(0, _RPW // 16, hid_body, 0)

    for ci in range(_RPW // _CHUNK):
        r0 = ci * _CHUNK
        pltpu.async_copy(hf_hbm.at[hidx_v.at[pl.ds(r0, _CHUNK)]],
                         hbuf, sem0).wait()

        for r in range(_CHUNK):
            val = plsc.load_gather(
                vals_v, [jnp.full((16,), pbase + r0 + r, jnp.int32)])

            def dbody(dd, _, r=r, val=val):
                hbuf[r, pl.ds(dd * 16, 16)] = hbuf[r, pl.ds(dd * 16, 16)] * val
                return 0

            lax.fori_loop(0, D // 16, dbody, 0)

        pltpu.sync_copy(hbuf, newh_hbm.at[pl.ds(base + r0, _CHUNK)])

    rbufs = (rbuf0, rbuf1)
    obufs = (obuf0, obuf1)
    sems = (sem0, sem1)
    osems = (semo0, semo1)
    nci = _RPW // _CHUNK
    tasks = [(g, o, ci) for g, o in ((g1_hbm, o1_hbm), (g2_hbm, o2_hbm))
             for ci in range(nci)]
    in_flight = [None, None]
    out_flight = [None, None]

    def start(t, slot):
        g_hbm, _, ci = tasks[t]
        return pltpu.async_copy(
            g_hbm.at[hidx_v.at[pl.ds(ci * _CHUNK, _CHUNK)]],
            rbufs[slot], sems[slot])

    in_flight[0] = start(0, 0)
    for t in range(len(tasks)):
        slot = t % 2
        if t + 1 < len(tasks):
            in_flight[1 - slot] = start(t + 1, 1 - slot)
        in_flight[slot].wait()
        rbuf = rbufs[slot]
        obuf = obufs[slot]
        if out_flight[slot] is not None:
            out_flight[slot].wait()

        def qbody(q, _, rbuf=rbuf, obuf=obuf):
            cols = idx_v[pl.ds(q * 16, 16)]
            for r in range(_CHUNK):
                rid = jnp.full((16,), r, jnp.int32)
                v = plsc.load_gather(rbuf, [rid, cols])
                obuf[r, pl.ds(q * 16, 16)] = v
            return 0

        lax.fori_loop(0, K // 16, qbody, 0)
        _, o_hbm, ci = tasks[t]
        out_flight[slot] = pltpu.async_copy(
            obuf, o_hbm.at[pl.ds(base + ci * _CHUNK, _CHUNK)], osems[slot])
    out_flight[0].wait()
    out_flight[1].wait()


@functools.cache
def _gather_sc_kernel():
    return functools.partial(
        pl.kernel,
        mesh=plsc.VectorSubcoreMesh(core_axis_name="c", subcore_axis_name="s"),
        out_type=[
            jax.ShapeDtypeStruct((B * K, K), jnp.float32),
            jax.ShapeDtypeStruct((B * K, K), jnp.float32),
            jax.ShapeDtypeStruct((B * K, D), jnp.float32),
        ],
        scratch_types=[
            pltpu.VMEM((N,), jnp.int32),
            pltpu.VMEM((N,), jnp.float32),
            pltpu.VMEM((K,), jnp.int32),
            pltpu.VMEM((K,), jnp.float32),
            pltpu.VMEM((_RPW,), jnp.int32),
            pltpu.VMEM((_CHUNK, D), jnp.float32),
            pltpu.VMEM((_CHUNK, N), jnp.float32),
            pltpu.VMEM((_CHUNK, N), jnp.float32),
            pltpu.VMEM((_CHUNK, K), jnp.float32),
            pltpu.VMEM((_CHUNK, K), jnp.float32),
            pltpu.SemaphoreType.DMA,
            pltpu.SemaphoreType.DMA,
            pltpu.SemaphoreType.DMA,
            pltpu.SemaphoreType.DMA,
        ],
        compiler_params=pltpu.CompilerParams(
            use_tc_tiling_on_sc=False, needs_layout_passes=False),
    )(_gather_body)


def kernel(g1, g2, h, section_feature):
    s, rank = _score_rank_tc(h, section_feature)
    g1f = g1.reshape(B * N, N)
    g2f = g2.reshape(B * N, N)
    hf = h.reshape(B * N, D)
    o1, o2, newh = _gather_sc_kernel()(g1f, g2f, hf, s, rank)
    return (o1.reshape(B, K, K), o2.reshape(B, K, K), newh.reshape(B, K, D))


# speedup vs baseline: 32.0602x; 1.6176x over previous
import functools

import jax
import jax.numpy as jnp
from jax import lax
from jax.experimental import pallas as pl
from jax.experimental.pallas import tpu as pltpu
from jax.experimental.pallas import tpu_sc as plsc

B = 4
N = 2048
K = 1024
D = 256
NSEC = 64


_RC = 128
_RNCH = N // _RC
_JC = 256


def _score_body(h_ref, sf_ref, s_ref):
    w = lax.dot_general(h_ref[0], sf_ref[0], (((1,), (1,)), ((), ())))
    s_ref[0, 0] = 1.0 / (1.0 + jnp.exp(-jnp.max(w, axis=1)))


def _rank_body(s_ref, schunk_ref, rank_ref):
    c = pl.program_id(1)
    sc = schunk_ref[0, 0].reshape(_RC, 1)
    icol = lax.broadcasted_iota(jnp.int32, (_RC, 1), 0) + c * _RC
    s_all = s_ref[0, 0]
    acc = jnp.zeros((_RC, _JC), jnp.float32)
    for jc in range(N // _JC):
        sj = s_all[jc * _JC:(jc + 1) * _JC].reshape(1, _JC)
        jrow = lax.broadcasted_iota(jnp.int32, (1, _JC), 1) + jc * _JC
        gt = (sj > sc).astype(jnp.float32)
        eqlt = ((sj == sc) & (jrow < icol)).astype(jnp.float32)
        acc = acc + gt + eqlt
    rank_ref[0, 0, 0] = jnp.sum(acc, axis=1).astype(jnp.int32)


def _score_rank_tc(h, section_feature):
    s = pl.pallas_call(
        _score_body,
        grid=(B,),
        in_specs=[
            pl.BlockSpec((1, N, D), lambda b: (b, 0, 0)),
            pl.BlockSpec((1, NSEC, D), lambda b: (b, 0, 0)),
        ],
        out_specs=pl.BlockSpec((1, 1, N), lambda b: (b, 0, 0)),
        out_shape=jax.ShapeDtypeStruct((B, 1, N), jnp.float32),
    )(h, section_feature)
    rank = pl.pallas_call(
        _rank_body,
        grid=(B, _RNCH),
        in_specs=[
            pl.BlockSpec((1, 1, N), lambda b, c: (b, 0, 0)),
            pl.BlockSpec((1, 1, _RC), lambda b, c: (b, 0, c)),
        ],
        out_specs=pl.BlockSpec((1, 1, 1, _RC), lambda b, c: (b, c, 0, 0)),
        out_shape=jax.ShapeDtypeStruct((B, _RNCH, 1, _RC), jnp.int32),
    )(s, s)
    return s.reshape(B * N), rank.reshape(B * N)


_NC = 2
_NS = 16
_NW = _NC * _NS
_RPW = (B * K) // _NW
_CHUNK = 16
_WPB = K // _RPW


def _gather_body(g1_hbm, g2_hbm, hf_hbm, s_hbm, rank_hbm,
                 o1_hbm, o2_hbm, newh_hbm,
                 rank_v, s_v, idx_v, vals_v, hidx_v, hbuf,
                 rbuf0, rbuf1, obuf0, obuf1, sem0, sem1, semo0, semo1):
    wid = lax.axis_index("s") * _NC + lax.axis_index("c")
    base = wid * _RPW
    b = wid // _WPB
    nodebase = pl.multiple_of(b * N, N)
    pbase = (wid % _WPB) * _RPW

    pltpu.sync_copy(rank_hbm.at[pl.ds(nodebase, N)], rank_v)
    pltpu.sync_copy(s_hbm.at[pl.ds(nodebase, N)], s_v)

    def inv_body(t, _):
        r16 = rank_v[pl.ds(t * 16, 16)]
        i16 = lax.broadcasted_iota(jnp.int32, (16,), 0) + t * 16
        m = r16 < K
        plsc.store_scatter(idx_v, [r16], i16, mask=m)
        plsc.store_scatter(vals_v, [r16], s_v[pl.ds(t * 16, 16)], mask=m)
        return 0

    lax.fori_loop(0, N // 16, inv_body, 0)

    def hid_body(t, _):
        hidx_v[pl.ds(t * 16, 16)] = (
            idx_v[pl.ds(pbase + t * 16, 16)] + nodebase)
        return 0

    lax.fori_loop(0, _RPW // 16, hid_body, 0)

    for ci in range(_RPW // _CHUNK):
        r0 = ci * _CHUNK
        pltpu.async_copy(hf_hbm.at[hidx_v.at[pl.ds(r0, _CHUNK)]],
                         hbuf, sem0).wait()

        for r in range(_CHUNK):
            val = plsc.load_gather(
                vals_v, [jnp.full((16,), pbase + r0 + r, jnp.int32)])

            def dbody(dd, _, r=r, val=val):
                hbuf[r, pl.ds(dd * 16, 16)] = hbuf[r, pl.ds(dd * 16, 16)] * val
                return 0

            lax.fori_loop(0, D // 16, dbody, 0)

        pltpu.sync_copy(hbuf, newh_hbm.at[pl.ds(base + r0, _CHUNK)])

    rbufs = (rbuf0, rbuf1)
    obufs = (obuf0, obuf1)
    sems = (sem0, sem1)
    osems = (semo0, semo1)
    nci = _RPW // _CHUNK
    tasks = [(g, o, ci) for g, o in ((g1_hbm, o1_hbm), (g2_hbm, o2_hbm))
             for ci in range(nci)]
    in_flight = [None, None]
    out_flight = [None, None]

    def start(t, slot):
        g_hbm, _, ci = tasks[t]
        return pltpu.async_copy(
            g_hbm.at[hidx_v.at[pl.ds(ci * _CHUNK, _CHUNK)]],
            rbufs[slot], sems[slot])

    in_flight[0] = start(0, 0)
    for t in range(len(tasks)):
        slot = t % 2
        if t + 1 < len(tasks):
            in_flight[1 - slot] = start(t + 1, 1 - slot)
        in_flight[slot].wait()
        rbuf = rbufs[slot]
        obuf = obufs[slot]
        if out_flight[slot] is not None:
            out_flight[slot].wait()

        def qbody(q, _, rbuf=rbuf, obuf=obuf):
            cols = idx_v[pl.ds(q * 16, 16)]
            for r in range(_CHUNK):
                rid = jnp.full((16,), r, jnp.int32)
                v = plsc.load_gather(rbuf, [rid, cols])
                obuf[r, pl.ds(q * 16, 16)] = v
            return 0

        lax.fori_loop(0, K // 16, qbody, 0)
        _, o_hbm, ci = tasks[t]
        out_flight[slot] = pltpu.async_copy(
            obuf, o_hbm.at[pl.ds(base + ci * _CHUNK, _CHUNK)], osems[slot])
    out_flight[0].wait()
    out_flight[1].wait()


@functools.cache
def _gather_sc_kernel():
    return functools.partial(
        pl.kernel,
        mesh=plsc.VectorSubcoreMesh(core_axis_name="c", subcore_axis_name="s"),
        out_type=[
            jax.ShapeDtypeStruct((B * K, K), jnp.float32),
            jax.ShapeDtypeStruct((B * K, K), jnp.float32),
            jax.ShapeDtypeStruct((B * K, D), jnp.float32),
        ],
        scratch_types=[
            pltpu.VMEM((N,), jnp.int32),
            pltpu.VMEM((N,), jnp.float32),
            pltpu.VMEM((K,), jnp.int32),
            pltpu.VMEM((K,), jnp.float32),
            pltpu.VMEM((_RPW,), jnp.int32),
            pltpu.VMEM((_CHUNK, D), jnp.float32),
            pltpu.VMEM((_CHUNK, N), jnp.float32),
            pltpu.VMEM((_CHUNK, N), jnp.float32),
            pltpu.VMEM((_CHUNK, K), jnp.float32),
            pltpu.VMEM((_CHUNK, K), jnp.float32),
            pltpu.SemaphoreType.DMA,
            pltpu.SemaphoreType.DMA,
            pltpu.SemaphoreType.DMA,
            pltpu.SemaphoreType.DMA,
        ],
        compiler_params=pltpu.CompilerParams(
            use_tc_tiling_on_sc=True, needs_layout_passes=False),
    )(_gather_body)


def kernel(g1, g2, h, section_feature):
    s, rank = _score_rank_tc(h, section_feature)
    g1f = g1.reshape(B * N, N)
    g2f = g2.reshape(B * N, N)
    hf = h.reshape(B * N, D)
    o1, o2, newh = _gather_sc_kernel()(g1f, g2f, hf, s, rank)
    return (o1.reshape(B, K, K), o2.reshape(B, K, K), newh.reshape(B, K, D))


# speedup vs baseline: 34.0416x; 1.0618x over previous
import functools

import jax
import jax.numpy as jnp
from jax import lax
from jax.experimental import pallas as pl
from jax.experimental.pallas import tpu as pltpu
from jax.experimental.pallas import tpu_sc as plsc

B = 4
N = 2048
K = 1024
D = 256
NSEC = 64


_RC = 128
_RNCH = N // _RC
_JC = 256


def _score_body(h_ref, sf_ref, s_ref):
    w = lax.dot_general(h_ref[0], sf_ref[0], (((1,), (1,)), ((), ())))
    s_ref[0, 0] = 1.0 / (1.0 + jnp.exp(-jnp.max(w, axis=1)))


def _rank_body(s_ref, schunk_ref, rank_ref):
    c = pl.program_id(1)
    sc = schunk_ref[0, 0].reshape(_RC, 1)
    icol = lax.broadcasted_iota(jnp.int32, (_RC, 1), 0) + c * _RC
    s_all = s_ref[0, 0]
    acc = jnp.zeros((_RC, _JC), jnp.float32)
    for jc in range(N // _JC):
        sj = s_all[jc * _JC:(jc + 1) * _JC].reshape(1, _JC)
        jrow = lax.broadcasted_iota(jnp.int32, (1, _JC), 1) + jc * _JC
        gt = (sj > sc).astype(jnp.float32)
        eqlt = ((sj == sc) & (jrow < icol)).astype(jnp.float32)
        acc = acc + gt + eqlt
    rank_ref[0, 0, 0] = jnp.sum(acc, axis=1).astype(jnp.int32)


def _score_rank_tc(h, section_feature):
    s = pl.pallas_call(
        _score_body,
        grid=(B,),
        in_specs=[
            pl.BlockSpec((1, N, D), lambda b: (b, 0, 0)),
            pl.BlockSpec((1, NSEC, D), lambda b: (b, 0, 0)),
        ],
        out_specs=pl.BlockSpec((1, 1, N), lambda b: (b, 0, 0)),
        out_shape=jax.ShapeDtypeStruct((B, 1, N), jnp.float32),
    )(h, section_feature)
    rank = pl.pallas_call(
        _rank_body,
        grid=(B, _RNCH),
        in_specs=[
            pl.BlockSpec((1, 1, N), lambda b, c: (b, 0, 0)),
            pl.BlockSpec((1, 1, _RC), lambda b, c: (b, 0, c)),
        ],
        out_specs=pl.BlockSpec((1, 1, 1, _RC), lambda b, c: (b, c, 0, 0)),
        out_shape=jax.ShapeDtypeStruct((B, _RNCH, 1, _RC), jnp.int32),
    )(s, s)
    return s.reshape(B * N), rank.reshape(B * N)


_NC = 2
_NS = 16
_NW = _NC * _NS
_RPW = (B * K) // _NW
_CHUNK = 16
_WPB = K // _RPW


def _gather_body(g1_hbm, g2_hbm, hf_hbm, s_hbm, rank_hbm,
                 o1_hbm, o2_hbm, newh_hbm,
                 rank_v, s_v, idx_v, vals_v, hidx_v, hbuf,
                 rbuf0, rbuf1, obuf0, obuf1, sem0, sem1, semo0, semo1, semh):
    wid = lax.axis_index("s") * _NC + lax.axis_index("c")
    base = wid * _RPW
    b = wid // _WPB
    nodebase = pl.multiple_of(b * N, N)
    pbase = (wid % _WPB) * _RPW

    cpr = pltpu.async_copy(rank_hbm.at[pl.ds(nodebase, N)], rank_v, sem0)
    cps = pltpu.async_copy(s_hbm.at[pl.ds(nodebase, N)], s_v, sem1)
    cpr.wait()
    cps.wait()

    def inv_body(t, _):
        r16 = rank_v[pl.ds(t * 16, 16)]
        i16 = lax.broadcasted_iota(jnp.int32, (16,), 0) + t * 16
        m = r16 < K
        plsc.store_scatter(idx_v, [r16], i16, mask=m)
        plsc.store_scatter(vals_v, [r16], s_v[pl.ds(t * 16, 16)], mask=m)
        return 0

    lax.fori_loop(0, N // 16, inv_body, 0)

    def hid_body(t, _):
        hidx_v[pl.ds(t * 16, 16)] = (
            idx_v[pl.ds(pbase + t * 16, 16)] + nodebase)
        return 0

    lax.fori_loop(0, _RPW // 16, hid_body, 0)

    rbufs = (rbuf0, rbuf1)
    obufs = (obuf0, obuf1)
    sems = (sem0, sem1)
    osems = (semo0, semo1)
    nci = _RPW // _CHUNK
    tasks = [(g, o, ci) for g, o in ((g1_hbm, o1_hbm), (g2_hbm, o2_hbm))
             for ci in range(nci)]
    in_flight = [None, None]
    out_flight = [None, None]

    def start(t, slot):
        g_hbm, _, ci = tasks[t]
        return pltpu.async_copy(
            g_hbm.at[hidx_v.at[pl.ds(ci * _CHUNK, _CHUNK)]],
            rbufs[slot], sems[slot])

    in_flight[0] = start(0, 0)

    _HB = _RPW // 2
    for hi in range(2):
        pltpu.async_copy(hf_hbm.at[hidx_v.at[pl.ds(hi * _HB, _HB)]],
                         hbuf, semh).wait()

        def hrow_body(r, _, hi=hi):
            val = plsc.load_gather(
                vals_v, [jnp.full((16,), pbase + hi * _HB + r, jnp.int32)])

            def dbody(dd, _, r=r, val=val):
                hbuf[r, pl.ds(dd * 16, 16)] = hbuf[r, pl.ds(dd * 16, 16)] * val
                return 0

            lax.fori_loop(0, D // 16, dbody, 0)
            return 0

        lax.fori_loop(0, _HB, hrow_body, 0)
        pltpu.sync_copy(hbuf, newh_hbm.at[pl.ds(base + hi * _HB, _HB)])

    for t in range(len(tasks)):
        slot = t % 2
        if t + 1 < len(tasks):
            in_flight[1 - slot] = start(t + 1, 1 - slot)
        in_flight[slot].wait()
        rbuf = rbufs[slot]
        obuf = obufs[slot]
        if out_flight[slot] is not None:
            out_flight[slot].wait()

        def qbody(q, _, rbuf=rbuf, obuf=obuf):
            cols = idx_v[pl.ds(q * 16, 16)]
            for r in range(_CHUNK):
                rid = jnp.full((16,), r, jnp.int32)
                v = plsc.load_gather(rbuf, [rid, cols])
                obuf[r, pl.ds(q * 16, 16)] = v
            return 0

        lax.fori_loop(0, K // 16, qbody, 0)
        _, o_hbm, ci = tasks[t]
        out_flight[slot] = pltpu.async_copy(
            obuf, o_hbm.at[pl.ds(base + ci * _CHUNK, _CHUNK)], osems[slot])
    out_flight[0].wait()
    out_flight[1].wait()


@functools.cache
def _gather_sc_kernel():
    return functools.partial(
        pl.kernel,
        mesh=plsc.VectorSubcoreMesh(core_axis_name="c", subcore_axis_name="s"),
        out_type=[
            jax.ShapeDtypeStruct((B * K, K), jnp.float32),
            jax.ShapeDtypeStruct((B * K, K), jnp.float32),
            jax.ShapeDtypeStruct((B * K, D), jnp.float32),
        ],
        scratch_types=[
            pltpu.VMEM((N,), jnp.int32),
            pltpu.VMEM((N,), jnp.float32),
            pltpu.VMEM((K,), jnp.int32),
            pltpu.VMEM((K,), jnp.float32),
            pltpu.VMEM((_RPW,), jnp.int32),
            pltpu.VMEM((_RPW // 2, D), jnp.float32),
            pltpu.VMEM((_CHUNK, N), jnp.float32),
            pltpu.VMEM((_CHUNK, N), jnp.float32),
            pltpu.VMEM((_CHUNK, K), jnp.float32),
            pltpu.VMEM((_CHUNK, K), jnp.float32),
            pltpu.SemaphoreType.DMA,
            pltpu.SemaphoreType.DMA,
            pltpu.SemaphoreType.DMA,
            pltpu.SemaphoreType.DMA,
            pltpu.SemaphoreType.DMA,
        ],
        compiler_params=pltpu.CompilerParams(
            use_tc_tiling_on_sc=True, needs_layout_passes=False),
    )(_gather_body)


def kernel(g1, g2, h, section_feature):
    s, rank = _score_rank_tc(h, section_feature)
    g1f = g1.reshape(B * N, N)
    g2f = g2.reshape(B * N, N)
    hf = h.reshape(B * N, D)
    o1, o2, newh = _gather_sc_kernel()(g1f, g2f, hf, s, rank)
    return (o1.reshape(B, K, K), o2.reshape(B, K, K), newh.reshape(B, K, D))
